# baseline (speedup 1.0000x reference)
import jax
import jax.numpy as jnp
from jax import lax
from jax.experimental import pallas as pl
from jax.experimental.pallas import tpu as pltpu

DIM = 192
NPOS = 196
NROW = 14
WIN = 208


def _body(s_ref, n_ref, nf_ref, r_ref, out_ref, buf, sem):
    b = pl.program_id(0)
    start = s_ref[b]
    n = n_ref[b]
    astart = (start // 8) * 8
    d = start - astart
    cp = pltpu.make_async_copy(nf_ref.at[pl.ds(astart, WIN), :], buf, sem)
    cp.start()
    cp.wait()
    w = buf[...]
    rmat = r_ref[d]
    g2 = jax.lax.dot_general(
        rmat, w, (((1,), (0,)), ((), ())), preferred_element_type=jnp.float32
    )
    j = lax.broadcasted_iota(jnp.int32, (NPOS, 1), 0)
    s = (j % NROW) * NROW + j // NROW
    g2 = jnp.where(s < n, g2, 0.0)
    out_ref[...] = g2.T.reshape(1, DIM, NPOS)


def kernel(node_features, num_sp_list):
    ns = num_sp_list.astype(jnp.int32)
    starts = (jnp.cumsum(ns) - ns).astype(jnp.int32)
    B = ns.shape[0]
    jj = jnp.arange(NPOS, dtype=jnp.int32)
    pj = (jj % NROW) * NROW + jj // NROW
    kk = jnp.arange(WIN, dtype=jnp.int32)
    rall = (
        (pj[None, :, None] + jnp.arange(8, dtype=jnp.int32)[:, None, None])
        == kk[None, None, :]
    ).astype(jnp.float32)
    grid_spec = pltpu.PrefetchScalarGridSpec(
        num_scalar_prefetch=2,
        grid=(B,),
        in_specs=[
            pl.BlockSpec(memory_space=pl.ANY),
            pl.BlockSpec((8, NPOS, WIN), lambda b, s_ref, n_ref: (0, 0, 0)),
        ],
        out_specs=pl.BlockSpec((1, DIM, NPOS), lambda b, s_ref, n_ref: (b, 0, 0)),
        scratch_shapes=[
            pltpu.VMEM((WIN, DIM), jnp.float32),
            pltpu.SemaphoreType.DMA,
        ],
    )
    out = pl.pallas_call(
        _body,
        grid_spec=grid_spec,
        out_shape=jax.ShapeDtypeStruct((B, DIM, NPOS), jnp.float32),
    )(starts, ns, node_features, rall)
    return out.reshape(B, DIM, NROW, NROW)


# speedup vs baseline: 3.0808x; 3.0808x over previous
import jax
import jax.numpy as jnp
from jax import lax
from jax.experimental import pallas as pl
from jax.experimental.pallas import tpu as pltpu

DIM = 192
NPOS = 196
NROW = 14
WIN = 208
NB = 8


def _body(s_ref, n_ref, nf_ref, rt_ref, out_ref, buf, sem):
    i = pl.program_id(0)
    ngroups = pl.num_programs(0)

    def issue(g, slot, wait):
        for n in range(NB):
            b = g * NB + n
            astart = (s_ref[b] // 8) * 8
            cp = pltpu.make_async_copy(
                nf_ref.at[pl.ds(astart, WIN), :], buf.at[slot, n], sem.at[slot, n]
            )
            if wait:
                cp.wait()
            else:
                cp.start()

    @pl.when(i == 0)
    def _():
        issue(0, 0, wait=False)

    @pl.when(i + 1 < ngroups)
    def _():
        issue(i + 1, (i + 1) % 2, wait=False)

    issue(i, i % 2, wait=True)

    slot = i % 2
    jlane = lax.broadcasted_iota(jnp.int32, (1, NPOS), 1)
    s_lane = (jlane % NROW) * NROW + jlane // NROW
    for n in range(NB):
        b = i * NB + n
        start = s_ref[b]
        d = start - (start // 8) * 8
        w = buf[slot, n]
        rt = rt_ref[d]
        o = lax.dot_general(
            w, rt, (((0,), (0,)), ((), ())), preferred_element_type=jnp.float32
        )
        o = o * (s_lane < n_ref[b]).astype(jnp.float32)
        out_ref[n] = o


def kernel(node_features, num_sp_list):
    ns = num_sp_list.astype(jnp.int32)
    starts = (jnp.cumsum(ns) - ns).astype(jnp.int32)
    B = ns.shape[0]
    jj = jnp.arange(NPOS, dtype=jnp.int32)
    pj = (jj % NROW) * NROW + jj // NROW
    kk = jnp.arange(WIN, dtype=jnp.int32)
    rtall = (
        kk[None, :, None]
        == (pj[None, None, :] + jnp.arange(8, dtype=jnp.int32)[:, None, None])
    ).astype(jnp.float32)
    grid_spec = pltpu.PrefetchScalarGridSpec(
        num_scalar_prefetch=2,
        grid=(B // NB,),
        in_specs=[
            pl.BlockSpec(memory_space=pl.ANY),
            pl.BlockSpec((8, WIN, NPOS), lambda i, s_ref, n_ref: (0, 0, 0)),
        ],
        out_specs=pl.BlockSpec((NB, DIM, NPOS), lambda i, s_ref, n_ref: (i, 0, 0)),
        scratch_shapes=[
            pltpu.VMEM((2, NB, WIN, DIM), jnp.float32),
            pltpu.SemaphoreType.DMA((2, NB)),
        ],
    )
    out = pl.pallas_call(
        _body,
        grid_spec=grid_spec,
        out_shape=jax.ShapeDtypeStruct((B, DIM, NPOS), jnp.float32),
    )(starts, ns, node_features, rtall)
    return out.reshape(B, DIM, NROW, NROW)


# speedup vs baseline: 3.3144x; 1.0758x over previous
import jax
import jax.numpy as jnp
from jax import lax
from jax.experimental import pallas as pl
from jax.experimental.pallas import tpu as pltpu

DIM = 192
NPOS = 196
NROW = 14
WIN = 208
NB = 8
_CHUNKS = ((0, 48), (48, 48), (96, 48), (144, 48), (192, 16))


def _body(s_ref, n_ref, nf_ref, rt_ref, out_ref, buf, sem):
    i = pl.program_id(0)
    ngroups = pl.num_programs(0)

    def issue(g, slot, wait):
        for n in range(NB):
            b = g * NB + n
            start = s_ref[b]
            astart = (start // 8) * 8
            need = start - astart + n_ref[b]
            for c0, clen in _CHUNKS:
                cp = pltpu.make_async_copy(
                    nf_ref.at[pl.ds(astart + c0, clen), :],
                    buf.at[slot, n, pl.ds(c0, clen)],
                    sem.at[slot, n],
                )

                @pl.when(jnp.logical_or(g < 2, jnp.logical_or(c0 == 0, c0 < need)))
                def _():
                    if wait:
                        cp.wait()
                    else:
                        cp.start()

    @pl.when(i == 0)
    def _():
        issue(0, 0, wait=False)

    @pl.when(i + 1 < ngroups)
    def _():
        issue(i + 1, (i + 1) % 2, wait=False)

    issue(i, i % 2, wait=True)

    slot = i % 2
    jlane = lax.broadcasted_iota(jnp.int32, (1, NPOS), 1)
    s_lane = (jlane % NROW) * NROW + jlane // NROW
    for n in range(NB):
        b = i * NB + n
        start = s_ref[b]
        d = start - (start // 8) * 8
        w = buf[slot, n]
        rt = rt_ref[d]
        o = lax.dot_general(
            w, rt, (((0,), (0,)), ((), ())), preferred_element_type=jnp.float32
        )
        o = o * (s_lane < n_ref[b]).astype(jnp.float32)
        out_ref[n] = o


def kernel(node_features, num_sp_list):
    ns = num_sp_list.astype(jnp.int32)
    starts = (jnp.cumsum(ns) - ns).astype(jnp.int32)
    B = ns.shape[0]
    jj = jnp.arange(NPOS, dtype=jnp.int32)
    pj = (jj % NROW) * NROW + jj // NROW
    kk = jnp.arange(WIN, dtype=jnp.int32)
    rtall = (
        kk[None, :, None]
        == (pj[None, None, :] + jnp.arange(8, dtype=jnp.int32)[:, None, None])
    ).astype(jnp.float32)
    grid_spec = pltpu.PrefetchScalarGridSpec(
        num_scalar_prefetch=2,
        grid=(B // NB,),
        in_specs=[
            pl.BlockSpec(memory_space=pl.ANY),
            pl.BlockSpec((8, WIN, NPOS), lambda i, s_ref, n_ref: (0, 0, 0)),
        ],
        out_specs=pl.BlockSpec((NB, DIM, NPOS), lambda i, s_ref, n_ref: (i, 0, 0)),
        scratch_shapes=[
            pltpu.VMEM((2, NB, WIN, DIM), jnp.float32),
            pltpu.SemaphoreType.DMA((2, NB)),
        ],
    )
    out = pl.pallas_call(
        _body,
        grid_spec=grid_spec,
        out_shape=jax.ShapeDtypeStruct((B, DIM, NPOS), jnp.float32),
    )(starts, ns, node_features, rtall)
    return out.reshape(B, DIM, NROW, NROW)


# speedup vs baseline: 4.3616x; 1.3160x over previous
import jax
import jax.numpy as jnp
from jax import lax
from jax.experimental import pallas as pl
from jax.experimental.pallas import tpu as pltpu

DIM = 192
NPOS = 196
NROW = 14
WINL = 384
NB = 8
_LCHUNKS = ((0, 128), (128, 128), (256, 128))


def _body(s_ref, n_ref, nft_ref, pm_ref, out_ref, buf, sem):
    i = pl.program_id(0)
    ngroups = pl.num_programs(0)

    def issue(g, slot, wait):
        for n in range(NB):
            b = g * NB + n
            start = s_ref[b]
            astart = (start // 128) * 128
            need = start - astart + n_ref[b]
            for c0, clen in _LCHUNKS:
                cp = pltpu.make_async_copy(
                    nft_ref.at[:, pl.ds(astart + c0, clen)],
                    buf.at[slot, n, :, pl.ds(c0, clen)],
                    sem.at[slot, n],
                )

                @pl.when(
                    jnp.logical_or(g < 2, jnp.logical_or(c0 == 0, c0 < need))
                )
                def _():
                    if wait:
                        cp.wait()
                    else:
                        cp.start()

    @pl.when(i == 0)
    def _():
        issue(0, 0, wait=False)

    @pl.when(i + 1 < ngroups)
    def _():
        issue(i + 1, (i + 1) % 2, wait=False)

    issue(i, i % 2, wait=True)

    slot = i % 2
    s_lane = lax.broadcasted_iota(jnp.int32, (1, NPOS), 1)
    pm = pm_ref[...]
    for n in range(NB):
        b = i * NB + n
        start = s_ref[b]
        o = start - (start // 128) * 128
        w = buf[slot, n]
        g1 = pltpu.roll(w, -o, axis=1)
        t = g1[:, :NPOS]
        t = t * (s_lane < n_ref[b]).astype(jnp.float32)
        o_b = lax.dot_general(
            t, pm, (((1,), (0,)), ((), ())), preferred_element_type=jnp.float32
        )
        out_ref[n] = o_b


def kernel(node_features, num_sp_list):
    ns = num_sp_list.astype(jnp.int32)
    starts = (jnp.cumsum(ns) - ns).astype(jnp.int32)
    B = ns.shape[0]
    nft = node_features.T
    jj = jnp.arange(NPOS, dtype=jnp.int32)
    pj = (jj % NROW) * NROW + jj // NROW
    pm = (jnp.arange(NPOS, dtype=jnp.int32)[:, None] == pj[None, :]).astype(
        jnp.float32
    )
    grid_spec = pltpu.PrefetchScalarGridSpec(
        num_scalar_prefetch=2,
        grid=(B // NB,),
        in_specs=[
            pl.BlockSpec(memory_space=pl.ANY),
            pl.BlockSpec((NPOS, NPOS), lambda i, s_ref, n_ref: (0, 0)),
        ],
        out_specs=pl.BlockSpec((NB, DIM, NPOS), lambda i, s_ref, n_ref: (i, 0, 0)),
        scratch_shapes=[
            pltpu.VMEM((2, NB, DIM, WINL), jnp.float32),
            pltpu.SemaphoreType.DMA((2, NB)),
        ],
    )
    out = pl.pallas_call(
        _body,
        grid_spec=grid_spec,
        out_shape=jax.ShapeDtypeStruct((B, DIM, NPOS), jnp.float32),
    )(starts, ns, nft, pm)
    return out.reshape(B, DIM, NROW, NROW)
